# baseline (speedup 1.0000x reference)
import functools

import jax
import jax.numpy as jnp
from jax import lax
from jax.experimental import pallas as pl
from jax.experimental.pallas import tpu as pltpu
from jax.experimental.pallas import tpu_sc as plsc

B = 4096
HIDDEN = 128
GAMMA = 12.0
NC = 2
NS = 16
NW = NC * NS
BPW = B // NW
CH = 32
NCH = BPW // CH
ND = HIDDEN // 16
TINY = 1e-35


def _rsqrt1(s):
    i = lax.bitcast_convert_type(s, jnp.int32)
    i = jnp.int32(0x5F3759DF) - (i >> 1)
    y = lax.bitcast_convert_type(i, jnp.float32)
    y = y * (1.5 - (0.5 * s) * y * y)
    return y


def _sc_body(*args):
    (hidx_hbm, ridx_hbm, tidx_hbm,
     ex_hbm, ey_hbm, ez_hbm,
     rw_hbm, rx_hbm, ry_hbm, rz_hbm,
     score_hbm, s1_hbm, s2_hbm, adx_hbm) = args[:14]
    hidx_v, ridx_v, tidx_v = args[14:17]
    gsets = (args[17:27], args[27:37])
    adx_bufs = args[37:39]
    sc_v, s1_v, s2_v = args[39:42]
    gsem, wsem = args[42:44]

    wid = lax.axis_index("s") * NC + lax.axis_index("c")
    base = wid * BPW
    pltpu.sync_copy(hidx_hbm.at[pl.ds(base, BPW)], hidx_v)
    pltpu.sync_copy(ridx_hbm.at[pl.ds(base, BPW)], ridx_v)
    pltpu.sync_copy(tidx_hbm.at[pl.ds(base, BPW)], tidx_v)

    def issue(c):
        bufs = gsets[c % 2]
        hs = hidx_v.at[pl.ds(c * CH, CH)]
        ts = tidx_v.at[pl.ds(c * CH, CH)]
        rs = ridx_v.at[pl.ds(c * CH, CH)]
        return [
            pltpu.async_copy(ex_hbm.at[hs], bufs[0], gsem),
            pltpu.async_copy(ey_hbm.at[hs], bufs[1], gsem),
            pltpu.async_copy(ez_hbm.at[hs], bufs[2], gsem),
            pltpu.async_copy(ex_hbm.at[ts], bufs[3], gsem),
            pltpu.async_copy(ey_hbm.at[ts], bufs[4], gsem),
            pltpu.async_copy(ez_hbm.at[ts], bufs[5], gsem),
            pltpu.async_copy(rw_hbm.at[rs], bufs[6], gsem),
            pltpu.async_copy(rx_hbm.at[rs], bufs[7], gsem),
            pltpu.async_copy(ry_hbm.at[rs], bufs[8], gsem),
            pltpu.async_copy(rz_hbm.at[rs], bufs[9], gsem),
        ]

    inflight = {0: issue(0), 1: issue(1)}
    pending_wb = [None, None]

    for c in range(NCH):
        for cp in inflight.pop(c):
            cp.wait()
        if pending_wb[c % 2] is not None:
            pending_wb[c % 2].wait()
        hx_v, hy_v, hz_v, tx_v, ty_v, tz_v, qw_v, qx_v, qy_v, qz_v = gsets[c % 2]
        adx_v = adx_bufs[c % 2]

        def row(r, carry):
            p_sc, p_s1 = carry

            def dblk(d4, a1):
              for dd in range(ND // 2):
                ds16 = pl.ds(d4 * (ND // 2 * 16) + dd * 16, 16)
                rw = qw_v[r, ds16]
                rx = qx_v[r, ds16]
                ry = qy_v[r, ds16]
                rz = qz_v[r, ds16]
                hx = hx_v[r, ds16]
                hy = hy_v[r, ds16]
                hz = hz_v[r, ds16]
                tx = tx_v[r, ds16]
                ty = ty_v[r, ds16]
                tz = tz_v[r, ds16]
                s = rw * rw + rx * rx + ry * ry + rz * rz
                k = 2.0 / jnp.maximum(s, TINY)
                kx = k * rx
                ky = k * ry
                kz = k * rz
                xx = kx * rx
                xy = kx * ry
                xz = kx * rz
                xw = kx * rw
                yy = ky * ry
                yz = ky * rz
                yw = ky * rw
                zz = kz * rz
                zw = kz * rw
                a11 = 1.0 - yy - zz
                a12 = xy - zw
                a13 = xz + yw
                a21 = xy + zw
                a22 = 1.0 - xx - zz
                a23 = yz - xw
                a31 = xz - yw
                a32 = yz + xw
                a33 = 1.0 - xx - yy
                dx = a11 * hx + a12 * hy + a13 * hz - tx
                dy = a21 * hx + a22 * hy + a23 * hz - ty
                dz = a31 * hx + a32 * hy + a33 * hz - tz
                q1 = dx * dx + dy * dy + dz * dz
                a1 = a1 + q1 * _rsqrt1(jnp.maximum(q1, TINY))
                adx_v[r, ds16] = jnp.abs(dx)
              return a1

            a1 = lax.fori_loop(0, 2, dblk, jnp.zeros((16,), jnp.float32))
            s1m = jnp.sum(a1) * (1.0 / HIDDEN)
            lane = r & 15
            m = lax.iota(jnp.int32, 16) == lane
            p_s1 = jnp.where(m, s1m, p_s1)
            p_sc = jnp.where(m, GAMMA - s1m, p_sc)
            g = pl.multiple_of(c * CH + (r & ~15), 16)
            sc_v[pl.ds(g, 16)] = p_sc
            s1_v[pl.ds(g, 16)] = p_s1
            s2_v[pl.ds(g, 16)] = p_s1
            return p_sc, p_s1

        zero16 = jnp.zeros((16,), jnp.float32)
        lax.fori_loop(0, CH, row, (zero16, zero16))

        pending_wb[c % 2] = pltpu.async_copy(
            adx_v, adx_hbm.at[pl.ds(base + c * CH, CH)], wsem)
        if c + 2 < NCH:
            inflight[c + 2] = issue(c + 2)

    pending_wb[0].wait()
    pending_wb[1].wait()
    pltpu.sync_copy(sc_v, score_hbm.at[pl.ds(base, BPW)])
    pltpu.sync_copy(s1_v, s1_hbm.at[pl.ds(base, BPW)])
    pltpu.sync_copy(s2_v, s2_hbm.at[pl.ds(base, BPW)])


_sc_call = functools.partial(
    pl.kernel,
    out_type=[
        jax.ShapeDtypeStruct((B,), jnp.float32),
        jax.ShapeDtypeStruct((B,), jnp.float32),
        jax.ShapeDtypeStruct((B,), jnp.float32),
        jax.ShapeDtypeStruct((B, HIDDEN), jnp.float32),
    ],
    mesh=plsc.VectorSubcoreMesh(core_axis_name="c", subcore_axis_name="s"),
    compiler_params=pltpu.CompilerParams(needs_layout_passes=False),
    scratch_types=(
        [pltpu.VMEM((BPW,), jnp.int32)] * 3
        + [pltpu.VMEM((CH, HIDDEN), jnp.float32)] * 20
        + [pltpu.VMEM((CH, HIDDEN), jnp.float32)] * 2
        + [pltpu.VMEM((BPW,), jnp.float32)] * 3
        + [pltpu.SemaphoreType.DMA, pltpu.SemaphoreType.DMA]
    ),
)(_sc_body)


def kernel(sample, entity_x, entity_y, entity_z,
           relation_w, relation_x, relation_y, relation_z):
    h_idx = sample[:, 0]
    r_idx = sample[:, 1]
    t_idx = sample[:, 2]
    score, s1, s2, adx = _sc_call(
        h_idx, r_idx, t_idx,
        entity_x, entity_y, entity_z,
        relation_w, relation_x, relation_y, relation_z,
    )
    return score[:, None], s1[:, None], s2[:, None], adx[:, None, :]


# speedup vs baseline: 1.2750x; 1.2750x over previous
import functools

import jax
import jax.numpy as jnp
from jax import lax
from jax.experimental import pallas as pl
from jax.experimental.pallas import tpu as pltpu
from jax.experimental.pallas import tpu_sc as plsc

B = 4096
HIDDEN = 128
GAMMA = 12.0
NC = 2
NS = 16
NW = NC * NS
BPW = B // NW
CH = 32
NCH = BPW // CH
ND = HIDDEN // 16
TINY = 1e-35


def _rsqrt1(s):
    i = lax.bitcast_convert_type(s, jnp.int32)
    i = jnp.int32(0x5F3759DF) - (i >> 1)
    y = lax.bitcast_convert_type(i, jnp.float32)
    y = y * (1.5 - (0.5 * s) * y * y)
    return y


def _sc_body(*args):
    (sample_hbm,
     ex_hbm, ey_hbm, ez_hbm,
     rw_hbm, rx_hbm, ry_hbm, rz_hbm,
     score_hbm, s1_hbm, s2_hbm, adx_hbm) = args[:12]
    samp_v = args[12]
    hidx_v, ridx_v, tidx_v = args[13:16]
    gsets = (args[16:26], args[26:36])
    adx_bufs = args[36:38]
    sc_v, s1_v, s2_v = args[38:41]
    gsem, wsem = args[41:43]

    wid = lax.axis_index("s") * NC + lax.axis_index("c")
    base = wid * BPW
    pltpu.sync_copy(sample_hbm.at[pl.ds(base, BPW)], samp_v)
    lanes = lax.iota(jnp.int32, 16)
    for g in range(BPW // 16):
        rows16 = g * 16 + lanes
        sl = pl.ds(g * 16, 16)
        hidx_v[sl] = plsc.load_gather(samp_v, [rows16, jnp.zeros((16,), jnp.int32)])
        ridx_v[sl] = plsc.load_gather(samp_v, [rows16, jnp.ones((16,), jnp.int32)])
        tidx_v[sl] = plsc.load_gather(samp_v, [rows16, jnp.full((16,), 2, jnp.int32)])

    def issue(c):
        bufs = gsets[c % 2]
        hs = hidx_v.at[pl.ds(c * CH, CH)]
        ts = tidx_v.at[pl.ds(c * CH, CH)]
        rs = ridx_v.at[pl.ds(c * CH, CH)]
        return [
            pltpu.async_copy(ex_hbm.at[hs], bufs[0], gsem),
            pltpu.async_copy(ey_hbm.at[hs], bufs[1], gsem),
            pltpu.async_copy(ez_hbm.at[hs], bufs[2], gsem),
            pltpu.async_copy(ex_hbm.at[ts], bufs[3], gsem),
            pltpu.async_copy(ey_hbm.at[ts], bufs[4], gsem),
            pltpu.async_copy(ez_hbm.at[ts], bufs[5], gsem),
            pltpu.async_copy(rw_hbm.at[rs], bufs[6], gsem),
            pltpu.async_copy(rx_hbm.at[rs], bufs[7], gsem),
            pltpu.async_copy(ry_hbm.at[rs], bufs[8], gsem),
            pltpu.async_copy(rz_hbm.at[rs], bufs[9], gsem),
        ]

    inflight = {0: issue(0), 1: issue(1)}
    pending_wb = [None, None]

    for c in range(NCH):
        for cp in inflight.pop(c):
            cp.wait()
        if pending_wb[c % 2] is not None:
            pending_wb[c % 2].wait()
        hx_v, hy_v, hz_v, tx_v, ty_v, tz_v, qw_v, qx_v, qy_v, qz_v = gsets[c % 2]
        adx_v = adx_bufs[c % 2]

        def row(r, carry):
            p_sc, p_s1 = carry
            a1 = jnp.zeros((16,), jnp.float32)
            for d in range(ND):
                ds16 = pl.ds(d * 16, 16)
                rw = qw_v[r, ds16]
                rx = qx_v[r, ds16]
                ry = qy_v[r, ds16]
                rz = qz_v[r, ds16]
                hx = hx_v[r, ds16]
                hy = hy_v[r, ds16]
                hz = hz_v[r, ds16]
                tx = tx_v[r, ds16]
                ty = ty_v[r, ds16]
                tz = tz_v[r, ds16]
                s = rw * rw + rx * rx + ry * ry + rz * rz
                k = 2.0 / jnp.maximum(s, TINY)
                kx = k * rx
                ky = k * ry
                kz = k * rz
                xx = kx * rx
                xy = kx * ry
                xz = kx * rz
                xw = kx * rw
                yy = ky * ry
                yz = ky * rz
                yw = ky * rw
                zz = kz * rz
                zw = kz * rw
                a11 = 1.0 - yy - zz
                a12 = xy - zw
                a13 = xz + yw
                a21 = xy + zw
                a22 = 1.0 - xx - zz
                a23 = yz - xw
                a31 = xz - yw
                a32 = yz + xw
                a33 = 1.0 - xx - yy
                dx = a11 * hx + a12 * hy + a13 * hz - tx
                dy = a21 * hx + a22 * hy + a23 * hz - ty
                dz = a31 * hx + a32 * hy + a33 * hz - tz
                q1 = dx * dx + dy * dy + dz * dz
                a1 = a1 + q1 * _rsqrt1(jnp.maximum(q1, TINY))
                adx_v[r, ds16] = jnp.abs(dx)
            s1m = jnp.sum(a1) * (1.0 / HIDDEN)
            lane = r & 15
            m = lax.iota(jnp.int32, 16) == lane
            p_s1 = jnp.where(m, s1m, p_s1)
            p_sc = jnp.where(m, GAMMA - s1m, p_sc)
            g = pl.multiple_of(c * CH + (r & ~15), 16)
            sc_v[pl.ds(g, 16)] = p_sc
            s1_v[pl.ds(g, 16)] = p_s1
            s2_v[pl.ds(g, 16)] = p_s1
            return p_sc, p_s1

        zero16 = jnp.zeros((16,), jnp.float32)
        lax.fori_loop(0, CH, row, (zero16, zero16))

        pending_wb[c % 2] = pltpu.async_copy(
            adx_v, adx_hbm.at[pl.ds(base + c * CH, CH)], wsem)
        if c + 2 < NCH:
            inflight[c + 2] = issue(c + 2)

    pending_wb[0].wait()
    pending_wb[1].wait()
    pltpu.sync_copy(sc_v, score_hbm.at[pl.ds(base, BPW)])
    pltpu.sync_copy(s1_v, s1_hbm.at[pl.ds(base, BPW)])
    pltpu.sync_copy(s2_v, s2_hbm.at[pl.ds(base, BPW)])


_sc_call = functools.partial(
    pl.kernel,
    out_type=[
        jax.ShapeDtypeStruct((B,), jnp.float32),
        jax.ShapeDtypeStruct((B,), jnp.float32),
        jax.ShapeDtypeStruct((B,), jnp.float32),
        jax.ShapeDtypeStruct((B, HIDDEN), jnp.float32),
    ],
    mesh=plsc.VectorSubcoreMesh(core_axis_name="c", subcore_axis_name="s"),
    compiler_params=pltpu.CompilerParams(needs_layout_passes=False),
    scratch_types=(
        [pltpu.VMEM((BPW, 3), jnp.int32)]
        + [pltpu.VMEM((BPW,), jnp.int32)] * 3
        + [pltpu.VMEM((CH, HIDDEN), jnp.float32)] * 20
        + [pltpu.VMEM((CH, HIDDEN), jnp.float32)] * 2
        + [pltpu.VMEM((BPW,), jnp.float32)] * 3
        + [pltpu.SemaphoreType.DMA, pltpu.SemaphoreType.DMA]
    ),
)(_sc_body)


def kernel(sample, entity_x, entity_y, entity_z,
           relation_w, relation_x, relation_y, relation_z):
    score, s1, s2, adx = _sc_call(
        sample,
        entity_x, entity_y, entity_z,
        relation_w, relation_x, relation_y, relation_z,
    )
    return score[:, None], s1[:, None], s2[:, None], adx[:, None, :]


# speedup vs baseline: 1.2828x; 1.0061x over previous
import functools

import jax
import jax.numpy as jnp
from jax import lax
from jax.experimental import pallas as pl
from jax.experimental.pallas import tpu as pltpu
from jax.experimental.pallas import tpu_sc as plsc

B = 4096
HIDDEN = 128
GAMMA = 12.0
NC = 2
NS = 16
NW = NC * NS
BPW = B // NW
CH = 32
NCH = BPW // CH
ND = HIDDEN // 16
TINY = 1e-35


def _rsqrt1(s):
    i = lax.bitcast_convert_type(s, jnp.int32)
    i = jnp.int32(0x5F3759DF) - (i >> 1)
    y = lax.bitcast_convert_type(i, jnp.float32)
    y = y * (1.5 - (0.5 * s) * y * y)
    return y


def _sc_body(*args):
    (hidx_hbm, ridx_hbm, tidx_hbm,
     ex_hbm, ey_hbm, ez_hbm,
     rw_hbm, rx_hbm, ry_hbm, rz_hbm,
     score_hbm, s1_hbm, s2_hbm, adx_hbm) = args[:14]
    hidx_v, ridx_v, tidx_v = args[14:17]
    gsets = (args[17:27], args[27:37])
    adx_bufs = args[37:39]
    sc_v, s1_v, s2_v = args[39:42]
    gsem, wsem = args[42:44]

    wid = lax.axis_index("s") * NC + lax.axis_index("c")
    base = wid * BPW
    pltpu.sync_copy(hidx_hbm.at[pl.ds(base, BPW)], hidx_v)
    pltpu.sync_copy(ridx_hbm.at[pl.ds(base, BPW)], ridx_v)
    pltpu.sync_copy(tidx_hbm.at[pl.ds(base, BPW)], tidx_v)

    def issue(c):
        bufs = gsets[c % 2]
        hs = hidx_v.at[pl.ds(c * CH, CH)]
        ts = tidx_v.at[pl.ds(c * CH, CH)]
        rs = ridx_v.at[pl.ds(c * CH, CH)]
        return [
            pltpu.async_copy(ex_hbm.at[hs], bufs[0], gsem),
            pltpu.async_copy(ey_hbm.at[hs], bufs[1], gsem),
            pltpu.async_copy(ez_hbm.at[hs], bufs[2], gsem),
            pltpu.async_copy(ex_hbm.at[ts], bufs[3], gsem),
            pltpu.async_copy(ey_hbm.at[ts], bufs[4], gsem),
            pltpu.async_copy(ez_hbm.at[ts], bufs[5], gsem),
            pltpu.async_copy(rw_hbm.at[rs], bufs[6], gsem),
            pltpu.async_copy(rx_hbm.at[rs], bufs[7], gsem),
            pltpu.async_copy(ry_hbm.at[rs], bufs[8], gsem),
            pltpu.async_copy(rz_hbm.at[rs], bufs[9], gsem),
        ]

    inflight = {0: issue(0), 1: issue(1)}
    pending_wb = [None, None]

    for c in range(NCH):
        for cp in inflight.pop(c):
            cp.wait()
        if pending_wb[c % 2] is not None:
            pending_wb[c % 2].wait()
        hx_v, hy_v, hz_v, tx_v, ty_v, tz_v, qw_v, qx_v, qy_v, qz_v = gsets[c % 2]
        adx_v = adx_bufs[c % 2]

        def row(r, carry):
            p_sc, p_s1 = carry
            a1 = jnp.zeros((16,), jnp.float32)
            for d in range(ND):
                ds16 = pl.ds(d * 16, 16)
                rw = qw_v[r, ds16]
                rx = qx_v[r, ds16]
                ry = qy_v[r, ds16]
                rz = qz_v[r, ds16]
                hx = hx_v[r, ds16]
                hy = hy_v[r, ds16]
                hz = hz_v[r, ds16]
                tx = tx_v[r, ds16]
                ty = ty_v[r, ds16]
                tz = tz_v[r, ds16]
                s = rw * rw + rx * rx + ry * ry + rz * rz
                k = 2.0 / jnp.maximum(s, TINY)
                kx = k * rx
                ky = k * ry
                kz = k * rz
                xx = kx * rx
                xy = kx * ry
                xz = kx * rz
                xw = kx * rw
                yy = ky * ry
                yz = ky * rz
                yw = ky * rw
                zz = kz * rz
                zw = kz * rw
                a11 = 1.0 - yy - zz
                a12 = xy - zw
                a13 = xz + yw
                a21 = xy + zw
                a22 = 1.0 - xx - zz
                a23 = yz - xw
                a31 = xz - yw
                a32 = yz + xw
                a33 = 1.0 - xx - yy
                dx = a11 * hx + a12 * hy + a13 * hz - tx
                dy = a21 * hx + a22 * hy + a23 * hz - ty
                dz = a31 * hx + a32 * hy + a33 * hz - tz
                q1 = dx * dx + dy * dy + dz * dz
                a1 = a1 + q1 * _rsqrt1(jnp.maximum(q1, TINY))
                adx_v[r, ds16] = jnp.abs(dx)
            s1m = jnp.sum(a1) * (1.0 / HIDDEN)
            lane = r & 15
            m = lax.iota(jnp.int32, 16) == lane
            p_s1 = jnp.where(m, s1m, p_s1)
            p_sc = jnp.where(m, GAMMA - s1m, p_sc)
            g = pl.multiple_of(c * CH + (r & ~15), 16)
            sc_v[pl.ds(g, 16)] = p_sc
            s1_v[pl.ds(g, 16)] = p_s1
            s2_v[pl.ds(g, 16)] = p_s1
            return p_sc, p_s1

        zero16 = jnp.zeros((16,), jnp.float32)
        lax.fori_loop(0, CH, row, (zero16, zero16))

        pending_wb[c % 2] = pltpu.async_copy(
            adx_v, adx_hbm.at[pl.ds(base + c * CH, CH)], wsem)
        if c + 2 < NCH:
            inflight[c + 2] = issue(c + 2)

    pending_wb[0].wait()
    pending_wb[1].wait()
    pltpu.sync_copy(sc_v, score_hbm.at[pl.ds(base, BPW)])
    pltpu.sync_copy(s1_v, s1_hbm.at[pl.ds(base, BPW)])
    pltpu.sync_copy(s2_v, s2_hbm.at[pl.ds(base, BPW)])


_sc_call = functools.partial(
    pl.kernel,
    out_type=[
        jax.ShapeDtypeStruct((B,), jnp.float32),
        jax.ShapeDtypeStruct((B,), jnp.float32),
        jax.ShapeDtypeStruct((B,), jnp.float32),
        jax.ShapeDtypeStruct((B, HIDDEN), jnp.float32),
    ],
    mesh=plsc.VectorSubcoreMesh(core_axis_name="c", subcore_axis_name="s"),
    compiler_params=pltpu.CompilerParams(needs_layout_passes=False),
    scratch_types=(
        [pltpu.VMEM((BPW,), jnp.int32)] * 3
        + [pltpu.VMEM((CH, HIDDEN), jnp.float32)] * 20
        + [pltpu.VMEM((CH, HIDDEN), jnp.float32)] * 2
        + [pltpu.VMEM((BPW,), jnp.float32)] * 3
        + [pltpu.SemaphoreType.DMA, pltpu.SemaphoreType.DMA]
    ),
)(_sc_body)


def kernel(sample, entity_x, entity_y, entity_z,
           relation_w, relation_x, relation_y, relation_z):
    h_idx = sample[:, 0]
    r_idx = sample[:, 1]
    t_idx = sample[:, 2]
    score, s1, s2, adx = _sc_call(
        h_idx, r_idx, t_idx,
        entity_x, entity_y, entity_z,
        relation_w, relation_x, relation_y, relation_z,
    )
    return score[:, None], s1[:, None], s2[:, None], adx[:, None, :]
